# Initial kernel scaffold; baseline (speedup 1.0000x reference)
import jax
import jax.numpy as jnp
from jax.experimental import pallas as pl


def kernel(x, edge_index, W1, b1, W2, b2, gru_Wih_0, gru_Whh_0, gru_bih_0, gru_bhh_0, wt_W_0, wt_b_0, gcn_b_0, gru_Wih_1, gru_Whh_1, gru_bih_1, gru_bhh_1, wt_W_1, wt_b_1, gcn_b_1, post1_W, post1_b, anom_W, anom_b):
    raise NotImplementedError("write your pallas kernel here")


# speedup vs baseline: 27.1838x; 27.1838x over previous
import functools
import jax
import jax.numpy as jnp
from jax import lax
from jax.experimental import pallas as pl
from jax.experimental.pallas import tpu as pltpu
from jax.experimental.pallas import tpu_sc as plsc

N = 100000
E = 1600000
NC = 2
NS = 16
NW = NC * NS
L = 16

E_PAD = 1605632
N_PADD = 100352
HALF = 50000
N_ACCH = 50016
CAP_ROWS = 800

ROW_BLK = 2000

_E_TILE_CH = 49
_N_TILE = N_PADD // NW
_N_GRP = _N_TILE // L
_A_TILE = N_ACCH // NS


def _leaky(v):
    return jnp.where(v >= 0, v, 0.01 * v)


def _rsqrt_newton(d):
    i = lax.bitcast_convert_type(d, jnp.int32)
    y = lax.bitcast_convert_type(jnp.int32(0x5F3759DF) - (i >> 1), jnp.float32)
    for _ in range(3):
        y = y * (1.5 - 0.5 * d * y * y)
    return y


def _sc_mesh():
    return plsc.VectorSubcoreMesh(core_axis_name="c", subcore_axis_name="s",
                                  num_cores=NC, num_subcores=NS)


_SC_PARAMS = pltpu.CompilerParams(use_tc_tiling_on_sc=False,
                                  needs_layout_passes=False)


def _deg_dis_body(dst_hbm, src_hbm,
                  dis_hbm, parts, slists, dlists, counts,
                  deg_loc, dbuf, sbuf, dstg, sstg, tmp, dacc, stage, cbuf):
    c = lax.axis_index("c")
    s = lax.axis_index("s")

    zeros16 = jnp.zeros((L,), jnp.float32)
    ones16 = jnp.ones((L,), jnp.float32)

    def _zero(i, _):
        deg_loc[pl.ds(i * L, L)] = zeros16
        return 0
    lax.fori_loop(0, N_PADD // L, _zero, 0)

    lo = c * HALF
    hi = lo + HALF + c * HALF

    chunk0 = s * _E_TILE_CH

    def _chunk(ci, carry):
        off, pos = carry
        pltpu.sync_copy(dst_hbm.at[chunk0 + ci], dbuf)
        pltpu.sync_copy(src_hbm.at[chunk0 + ci], sbuf)

        def _grp(i, carry):
            off, pos = carry
            dv = dbuf[pl.ds(i * L, L)]
            sv = sbuf[pl.ds(i * L, L)]
            plsc.addupdate_scatter(deg_loc, [dv], ones16)
            m = (dv >= lo) & (dv < hi)
            plsc.store_compressed(dstg.at[pl.ds(off, L)], dv - lo, mask=m)
            plsc.store_compressed(sstg.at[pl.ds(off, L)], sv, mask=m)
            cnt = plsc.all_reduce_population_count(m)[0]
            off = off + cnt

            def _flush():
                for k in range(16):
                    pltpu.sync_copy(dstg.at[pl.ds(k * 128, 128)],
                                    dlists.at[c, s, pos + k])
                    pltpu.sync_copy(sstg.at[pl.ds(k * 128, 128)],
                                    slists.at[c, s, pos + k])
                dstg[pl.ds(0, L)] = dstg[pl.ds(2048, L)]
                sstg[pl.ds(0, L)] = sstg[pl.ds(2048, L)]

            do = off >= 2048
            pl.when(do)(_flush)
            off = jnp.where(do, off - 2048, off)
            pos = jnp.where(do, pos + 16, pos)
            return off, pos

        return lax.fori_loop(0, 2048 // L, _grp, (off, pos))

    off, pos = lax.fori_loop(0, _E_TILE_CH, _chunk,
                             (jnp.int32(0), jnp.int32(0)))

    dummy_d = jnp.full((L,), HALF, jnp.int32)
    dummy_s = jnp.zeros((L,), jnp.int32)

    def _pad(i, _):
        dstg[pl.ds(off + i * L, L)] = dummy_d
        sstg[pl.ds(off + i * L, L)] = dummy_s
        return 0
    lax.fori_loop(0, (2048 - off + L - 1) // L, _pad, 0)

    def _final_flush():
        for k in range(16):
            pltpu.sync_copy(dstg.at[pl.ds(k * 128, 128)],
                            dlists.at[c, s, pos + k])
            pltpu.sync_copy(sstg.at[pl.ds(k * 128, 128)],
                            slists.at[c, s, pos + k])
    pl.when(off > 0)(_final_flush)
    pos = jnp.where(off > 0, pos + 16, pos)

    nblk = pos >> 4
    iota = lax.iota(jnp.int32, L)
    cbuf[...] = jnp.where(iota == 0, nblk, 0)
    pltpu.sync_copy(cbuf, counts.at[c, s])

    pltpu.sync_copy(deg_loc, parts.at[c, s])
    plsc.subcore_barrier()

    w = c * NS + s
    nbase = w * _N_TILE

    def _zacc(i, _):
        dacc[pl.ds(i * L, L)] = zeros16
        return 0
    lax.fori_loop(0, _N_GRP, _zacc, 0)

    for t in range(NS):
        pltpu.sync_copy(parts.at[c, t, pl.ds(nbase, _N_TILE)], tmp)

        def _acc(i, _):
            dacc[pl.ds(i * L, L)] = dacc[pl.ds(i * L, L)] + tmp[pl.ds(i * L, L)]
            return 0
        lax.fori_loop(0, _N_GRP, _acc, 0)

    lane_ids = [jnp.full((L,), j, jnp.int32) for j in range(L)]
    n_stage = 24

    for part in range(9):
        glo = part * n_stage
        gn = n_stage if part < 8 else _N_GRP - 8 * n_stage
        for gi in range(gn):
            g = glo + gi
            d = dacc[pl.ds(g * L, L)] + 1.0
            y = _rsqrt_newton(d)
            for j in range(L):
                row = jnp.take_along_axis(y, lane_ids[j], axis=0)
                stage[gi * L + j, :] = row
        pltpu.sync_copy(stage.at[pl.ds(0, gn * L)],
                        dis_hbm.at[pl.ds(nbase + glo * L, gn * L)])


def _make_deg_dis():
    return functools.partial(
        pl.kernel,
        out_type=[
            jax.ShapeDtypeStruct((N_PADD, L), jnp.float32),
            jax.ShapeDtypeStruct((NC, NS, N_PADD), jnp.float32),
            jax.ShapeDtypeStruct((NC, NS, CAP_ROWS, 128), jnp.int32),
            jax.ShapeDtypeStruct((NC, NS, CAP_ROWS, 128), jnp.int32),
            jax.ShapeDtypeStruct((NC, NS, L), jnp.int32),
        ],
        mesh=_sc_mesh(),
        compiler_params=_SC_PARAMS,
        scratch_types=[
            pltpu.VMEM((N_PADD,), jnp.float32),
            pltpu.VMEM((2048,), jnp.int32),
            pltpu.VMEM((2048,), jnp.int32),
            pltpu.VMEM((2080,), jnp.int32),
            pltpu.VMEM((2080,), jnp.int32),
            pltpu.VMEM((_N_TILE,), jnp.float32),
            pltpu.VMEM((_N_TILE,), jnp.float32),
            pltpu.VMEM((24 * L, L), jnp.float32),
            pltpu.VMEM((L,), jnp.int32),
        ],
    )(_deg_dis_body)


def _scatter_body(hw2_hbm, slists, dlists, counts, accp_hbm,
                  sidx, didx, rows, zbuf, bounce, cbuf, acc_sh, gsem, ssem):
    c = lax.axis_index("c")
    s = lax.axis_index("s")

    zeros16 = jnp.zeros((L,), jnp.float32)

    def _zrow(i, _):
        zbuf[i, :] = zeros16
        return 0
    lax.fori_loop(0, 128, _zrow, 0)
    abase = s * _A_TILE
    for z in range(24):
        pltpu.sync_copy(zbuf, acc_sh.at[pl.ds(abase + z * 128, 128)])
    pltpu.sync_copy(zbuf.at[pl.ds(0, _A_TILE - 24 * 128)],
                    acc_sh.at[pl.ds(abase + 24 * 128, _A_TILE - 24 * 128)])
    plsc.subcore_barrier()

    pltpu.sync_copy(counts.at[c, s], cbuf)
    nblk = cbuf[pl.ds(0, L)][0]

    def _blk(b, _):
        pltpu.sync_copy(slists.at[c, s, pl.ds(b * L, L)], sidx)
        pltpu.sync_copy(dlists.at[c, s, pl.ds(b * L, L)], didx)
        gds = [pltpu.async_copy(hw2_hbm.at[sidx.at[k]], rows.at[k], gsem)
               for k in range(L)]
        for k in range(L):
            gds[k].wait()
        sds = [pltpu.async_copy(rows.at[k], acc_sh.at[didx.at[k]], ssem,
                                add=True) for k in range(L)]
        for k in range(L):
            sds[k].wait()
        return 0

    lax.fori_loop(0, nblk, _blk, 0)
    plsc.subcore_barrier()

    for z in range(6):
        pltpu.sync_copy(acc_sh.at[pl.ds(abase + z * 512, 512)], bounce)
        pltpu.sync_copy(bounce, accp_hbm.at[c, pl.ds(abase + z * 512, 512)])
    tail = _A_TILE - 6 * 512
    pltpu.sync_copy(acc_sh.at[pl.ds(abase + 6 * 512, tail)],
                    bounce.at[pl.ds(0, tail)])
    pltpu.sync_copy(bounce.at[pl.ds(0, tail)],
                    accp_hbm.at[c, pl.ds(abase + 6 * 512, tail)])


def _make_scatter():
    return functools.partial(
        pl.kernel,
        out_type=jax.ShapeDtypeStruct((NC, N_ACCH, L), jnp.float32),
        mesh=_sc_mesh(),
        compiler_params=_SC_PARAMS,
        scratch_types=[
            pltpu.VMEM((L, 128), jnp.int32),
            pltpu.VMEM((L, 128), jnp.int32),
            pltpu.VMEM((L, 128, L), jnp.float32),
            pltpu.VMEM((128, L), jnp.float32),
            pltpu.VMEM((512, L), jnp.float32),
            pltpu.VMEM((L,), jnp.int32),
            pltpu.VMEM_SHARED((N_ACCH, L), jnp.float32),
            pltpu.SemaphoreType.DMA,
            pltpu.SemaphoreType.DMA,
        ],
    )(_scatter_body)


def _acc_spec():
    return pl.BlockSpec((1, ROW_BLK, L), lambda i: (i // 25, i % 25, 0))


def _mlp_body(x_ref, dis_ref, w1_ref, b1_ref, w2_ref, b2_ref, wg_ref,
              hw2_ref):
    t = _leaky(jnp.dot(x_ref[...], w1_ref[...],
                       preferred_element_type=jnp.float32) + b1_ref[...])
    h = _leaky(jnp.dot(t, w2_ref[...],
                       preferred_element_type=jnp.float32) + b2_ref[...])
    hw2_ref[...] = jnp.dot(h, wg_ref[...],
                           preferred_element_type=jnp.float32) * dis_ref[...]


def _tc_mlp(x, dis16, w1t, b1, w2t, b2, wgt0):
    g = N // ROW_BLK
    return pl.pallas_call(
        _mlp_body,
        grid=(g,),
        in_specs=[
            pl.BlockSpec((ROW_BLK, 128), lambda i: (i, 0)),
            pl.BlockSpec((ROW_BLK, L), lambda i: (i, 0)),
            pl.BlockSpec((128, 256), lambda i: (0, 0)),
            pl.BlockSpec((1, 256), lambda i: (0, 0)),
            pl.BlockSpec((256, L), lambda i: (0, 0)),
            pl.BlockSpec((1, L), lambda i: (0, 0)),
            pl.BlockSpec((L, L), lambda i: (0, 0)),
        ],
        out_specs=pl.BlockSpec((ROW_BLK, L), lambda i: (i, 0)),
        out_shape=jax.ShapeDtypeStruct((N, L), jnp.float32),
    )(x, dis16, w1t, b1, w2t, b2, wgt0)


def _mid_body(acc_ref, hw2_ref, dis_ref, gb_ref, wg_ref, out_ref):
    a = acc_ref[0]
    h = _leaky(dis_ref[...] * (a + hw2_ref[...]) + gb_ref[...])
    out_ref[...] = jnp.dot(h, wg_ref[...],
                           preferred_element_type=jnp.float32) * dis_ref[...]


def _tc_mid(accp, hw2, dis16, gb, wgt1):
    g = N // ROW_BLK
    return pl.pallas_call(
        _mid_body,
        grid=(g,),
        in_specs=[
            _acc_spec(),
            pl.BlockSpec((ROW_BLK, L), lambda i: (i, 0)),
            pl.BlockSpec((ROW_BLK, L), lambda i: (i, 0)),
            pl.BlockSpec((1, L), lambda i: (0, 0)),
            pl.BlockSpec((L, L), lambda i: (0, 0)),
        ],
        out_specs=pl.BlockSpec((ROW_BLK, L), lambda i: (i, 0)),
        out_shape=jax.ShapeDtypeStruct((N, L), jnp.float32),
    )(accp, hw2, dis16, gb, wgt1)


def _fin_body(acc_ref, hw2_ref, dis_ref, gb_ref, p_ref, pb_ref,
              h_ref, o2_ref):
    a = acc_ref[0]
    h = _leaky(dis_ref[...] * (a + hw2_ref[...]) + gb_ref[...])
    h_ref[...] = h
    o2_ref[...] = jnp.dot(h, p_ref[...],
                          preferred_element_type=jnp.float32) + pb_ref[...]


def _tc_fin(accp, hw2, dis16, gb, p, pb):
    g = N // ROW_BLK
    return pl.pallas_call(
        _fin_body,
        grid=(g,),
        in_specs=[
            _acc_spec(),
            pl.BlockSpec((ROW_BLK, L), lambda i: (i, 0)),
            pl.BlockSpec((ROW_BLK, L), lambda i: (i, 0)),
            pl.BlockSpec((1, L), lambda i: (0, 0)),
            pl.BlockSpec((L, 8), lambda i: (0, 0)),
            pl.BlockSpec((1, 8), lambda i: (0, 0)),
        ],
        out_specs=[
            pl.BlockSpec((ROW_BLK, L), lambda i: (i, 0)),
            pl.BlockSpec((ROW_BLK, 8), lambda i: (i, 0)),
        ],
        out_shape=[
            jax.ShapeDtypeStruct((N, L), jnp.float32),
            jax.ShapeDtypeStruct((N, 8), jnp.float32),
        ],
    )(accp, hw2, dis16, gb, p, pb)


def _evolved_weight_t(bih, bhh, wtW, wtb):
    i_r, i_z, i_n = jnp.split(bih, 3)
    h_r, h_z, h_n = jnp.split(bhh, 3)
    r = jax.nn.sigmoid(i_r + h_r)
    z = jax.nn.sigmoid(i_z + h_z)
    nn_ = jnp.tanh(i_n + r * h_n)
    h1 = (1.0 - z) * nn_
    wg = (h1 @ wtW.T + wtb).reshape(L, L)
    return wg.T


def kernel(x, edge_index, W1, b1, W2, b2,
           gru_Wih_0, gru_Whh_0, gru_bih_0, gru_bhh_0, wt_W_0, wt_b_0,
           gcn_b_0,
           gru_Wih_1, gru_Whh_1, gru_bih_1, gru_bhh_1, wt_W_1, wt_b_1,
           gcn_b_1,
           post1_W, post1_b, anom_W, anom_b):
    src = edge_index[0]
    dst = edge_index[1]

    pad = E_PAD - E
    srcp = jnp.concatenate([src, jnp.zeros((pad,), jnp.int32)])
    dstp = jnp.concatenate([dst, jnp.full((pad,), N, jnp.int32)])
    src_ch = srcp.reshape(E_PAD // 2048, 2048)
    dst_ch = dstp.reshape(E_PAD // 2048, 2048)

    wgt0 = _evolved_weight_t(gru_bih_0, gru_bhh_0, wt_W_0, wt_b_0)
    wgt1 = _evolved_weight_t(gru_bih_1, gru_bhh_1, wt_W_1, wt_b_1)

    dis_full, _, slists, dlists, counts = _make_deg_dis()(dst_ch, src_ch)
    dis16 = dis_full[:N]

    hw2_0 = _tc_mlp(x, dis16, W1.T, b1.reshape(1, 256), W2.T,
                    b2.reshape(1, L), wgt0)

    accp0 = _make_scatter()(hw2_0, slists, dlists, counts)
    hw2_1 = _tc_mid(accp0, hw2_0, dis16, gcn_b_0.reshape(1, L), wgt1)

    accp1 = _make_scatter()(hw2_1, slists, dlists, counts)

    p = jnp.zeros((L, 8), jnp.float32)
    p = p.at[:, 0].set(post1_W[0] + post1_W[1])
    p = p.at[:, 1].set(anom_W[0])
    pb = jnp.zeros((1, 8), jnp.float32)
    pb = pb.at[0, 0].set(post1_b[0] + post1_b[1])
    pb = pb.at[0, 1].set(anom_b[0])

    h2, o2 = _tc_fin(accp1, hw2_1, dis16, gcn_b_1.reshape(1, L), p, pb)
    return o2[:, 0], o2[:, 1], h2
